# baseline (speedup 1.0000x reference)
import functools

import jax
import jax.numpy as jnp
from jax import lax
from jax.experimental import pallas as pl
from jax.experimental.pallas import tpu as pltpu
from jax.experimental.pallas import tpu_sc as plsc

_INFO = plsc.get_sparse_core_info()
_NC, _NS = _INFO.num_cores, _INFO.num_subcores
_NW = _NC * _NS

_CHUNK = 512
_NBUF = 4
_DEPTH = 2
_LEAD = 2


@functools.partial(jax.jit, static_argnames=("n_rows", "d"))
def _sc_gather(embeddings, idx, n_rows, d):
    rows_per_w = n_rows // _NW
    n_chunks = rows_per_w // _CHUNK
    mesh = plsc.VectorSubcoreMesh(core_axis_name="c", subcore_axis_name="s")

    @functools.partial(
        pl.kernel,
        mesh=mesh,
        out_type=jax.ShapeDtypeStruct((n_rows, d), jnp.float32),
        scratch_types=[
            [pltpu.VMEM((_CHUNK,), jnp.int32) for _ in range(_NBUF)],
            [pltpu.VMEM((_CHUNK, d), jnp.float32) for _ in range(_NBUF)],
            [pltpu.SemaphoreType.DMA for _ in range(_NBUF)],
            [pltpu.SemaphoreType.DMA for _ in range(_NBUF)],
            [pltpu.SemaphoreType.DMA for _ in range(_NBUF)],
        ],
        compiler_params=pltpu.CompilerParams(use_tc_tiling_on_sc=False),
    )
    def k(table_hbm, idx_hbm, out_hbm, idx_v, rows_v, sem_i, sem_g, sem_w):
        wid = lax.axis_index("s") * _NC + lax.axis_index("c")
        base = wid * rows_per_w

        idxcp = [None] * _NBUF
        gathers = [None] * _NBUF
        writes = [None] * _NBUF

        def stage_idx(c):
            b = c % _NBUF
            idxcp[b] = pltpu.async_copy(
                idx_hbm.at[pl.ds(base + c * _CHUNK, _CHUNK)], idx_v[b], sem_i[b]
            )

        def retire(c):
            pb = c % _NBUF
            gathers[pb].wait()
            writes[pb] = pltpu.async_copy(
                rows_v[pb],
                out_hbm.at[pl.ds(base + c * _CHUNK, _CHUNK)],
                sem_w[pb],
            )

        for c in range(min(_LEAD, n_chunks)):
            stage_idx(c)
        for c in range(n_chunks):
            b = c % _NBUF
            if c >= _DEPTH:
                retire(c - _DEPTH)
            if c + _LEAD < n_chunks:
                stage_idx(c + _LEAD)
            idxcp[b].wait()
            if writes[b] is not None:
                writes[b].wait()
                writes[b] = None
            gathers[b] = pltpu.async_copy(
                table_hbm.at[idx_v[b]], rows_v[b], sem_g[b]
            )
        for c in range(max(0, n_chunks - _DEPTH), n_chunks):
            retire(c)
        for b in range(_NBUF):
            if writes[b] is not None:
                writes[b].wait()

    return k(embeddings, idx)


def kernel(tokens, embeddings):
    b, s = tokens.shape
    v, d = embeddings.shape
    n_rows = b * s
    idx = tokens.reshape(n_rows).astype(jnp.int32)
    out = _sc_gather(embeddings, idx, n_rows, d)
    return out.reshape(b, s, d)


# speedup vs baseline: 1.0008x; 1.0008x over previous
import functools

import jax
import jax.numpy as jnp
from jax import lax
from jax.experimental import pallas as pl
from jax.experimental.pallas import tpu as pltpu
from jax.experimental.pallas import tpu_sc as plsc

_INFO = plsc.get_sparse_core_info()
_NC, _NS = _INFO.num_cores, _INFO.num_subcores
_NW = _NC * _NS

_CHUNK = 1024
_NBUF = 3


@functools.partial(jax.jit, static_argnames=("n_rows", "d"))
def _sc_gather(embeddings, idx, n_rows, d):
    rows_per_w = n_rows // _NW
    n_chunks = rows_per_w // _CHUNK
    mesh = plsc.VectorSubcoreMesh(core_axis_name="c", subcore_axis_name="s")

    @functools.partial(
        pl.kernel,
        mesh=mesh,
        out_type=jax.ShapeDtypeStruct((n_rows, d), jnp.float32),
        scratch_types=[
            pltpu.VMEM((rows_per_w,), jnp.int32),
            [pltpu.VMEM((_CHUNK, d), jnp.float32) for _ in range(_NBUF)],
            [pltpu.SemaphoreType.DMA for _ in range(_NBUF)],
            [pltpu.SemaphoreType.DMA for _ in range(_NBUF)],
        ],
        compiler_params=pltpu.CompilerParams(use_tc_tiling_on_sc=False),
    )
    def k(table_hbm, idx_hbm, out_hbm, idx_v, rows_v, sem_g, sem_w):
        wid = lax.axis_index("s") * _NC + lax.axis_index("c")
        base = wid * rows_per_w
        pltpu.sync_copy(idx_hbm.at[pl.ds(base, rows_per_w)], idx_v)

        depth = _NBUF - 1
        gathers = [None] * _NBUF
        writes = [None] * _NBUF

        def retire(c):
            pb = c % _NBUF
            gathers[pb].wait()
            writes[pb] = pltpu.async_copy(
                rows_v[pb],
                out_hbm.at[pl.ds(base + c * _CHUNK, _CHUNK)],
                sem_w[pb],
            )

        for c in range(n_chunks):
            b = c % _NBUF
            if writes[b] is not None:
                writes[b].wait()
                writes[b] = None
            gathers[b] = pltpu.async_copy(
                table_hbm.at[idx_v.at[pl.ds(c * _CHUNK, _CHUNK)]],
                rows_v[b],
                sem_g[b],
            )
            if c >= depth:
                retire(c - depth)
        for c in range(max(0, n_chunks - depth), n_chunks):
            retire(c)
        for b in range(_NBUF):
            if writes[b] is not None:
                writes[b].wait()

    return k(embeddings, idx)


def kernel(tokens, embeddings):
    b, s = tokens.shape
    v, d = embeddings.shape
    n_rows = b * s
    idx = tokens.reshape(n_rows).astype(jnp.int32)
    out = _sc_gather(embeddings, idx, n_rows, d)
    return out.reshape(b, s, d)
